# Initial kernel scaffold; baseline (speedup 1.0000x reference)
import jax
import jax.numpy as jnp
from jax.experimental import pallas as pl


def kernel(fused_obs, phase_embed, skill_latent, p_hat, beta, Wc, bc, W1, b1, W2, b2, W3, b3, Wd, bd):
    raise NotImplementedError("write your pallas kernel here")


# speedup vs baseline: 1.7050x; 1.7050x over previous
import functools

import jax
import jax.numpy as jnp
from jax.experimental import pallas as pl
from jax.experimental.pallas import tpu as pltpu

_K = 8
_LATENT = 128
_HID = 128
_STEPS = 4
_TA = 16
_DA = 32
_BT = 512


def _moe_body(x_ref, gate_ref, Wc_ref, bc_ref, W1u_ref, W1c_ref, w1tau_ref,
              b1_ref, W2_ref, b2_ref, W3_ref, b3_ref, Wd_ref, bd_ref,
              out_ref):
    f32 = jnp.float32
    x = x_ref[...]
    gate = gate_ref[...]

    cond = jnp.dot(x, Wc_ref[...], preferred_element_type=f32) + bc_ref[...]
    cp = jnp.dot(cond, W1c_ref[...], preferred_element_type=f32) + b1_ref[...]
    gb3 = jnp.dot(gate, b3_ref[...], preferred_element_type=f32)

    dt = 1.0 / _STEPS
    u = jnp.zeros((x.shape[0], _LATENT), f32)
    for i in range(_STEPS):
        pre = cp if i == 0 else (
            jnp.dot(u, W1u_ref[...], preferred_element_type=f32)
            + cp + (i * dt) * w1tau_ref[...])
        h1 = pre * jax.nn.sigmoid(pre)
        v = gb3
        for k in range(_K):
            h1k = h1[:, k * _HID:(k + 1) * _HID]
            a2 = jnp.dot(h1k, W2_ref[k], preferred_element_type=f32) + b2_ref[k]
            h2k = a2 * jax.nn.sigmoid(a2)
            v = v + jnp.dot(h2k * gate[:, k:k + 1], W3_ref[k],
                            preferred_element_type=f32)
        u = u + dt * v

    out_ref[...] = jnp.dot(u, Wd_ref[...], preferred_element_type=f32) + bd_ref[...]


@jax.jit
def kernel(fused_obs, phase_embed, skill_latent, p_hat, beta, Wc, bc, W1, b1,
           W2, b2, W3, b3, Wd, bd):
    del beta
    b = fused_obs.shape[0]
    x_in = jnp.concatenate([fused_obs, phase_embed, skill_latent], axis=-1)
    cond_in = x_in.shape[1]
    out_dim = Wd.shape[1]

    W1u = jnp.transpose(W1[:, :_LATENT, :], (1, 0, 2)).reshape(_LATENT, _K * _HID)
    W1c = jnp.transpose(W1[:, _LATENT:-1, :], (1, 0, 2)).reshape(-1, _K * _HID)
    w1tau = W1[:, -1, :].reshape(1, _K * _HID)
    b1f = b1.reshape(1, _K * _HID)

    grid = (b // _BT,)
    full = lambda *s: pl.BlockSpec(s, lambda i: (0,) * len(s))

    out = pl.pallas_call(
        _moe_body,
        grid=grid,
        in_specs=[
            pl.BlockSpec((_BT, cond_in), lambda i: (i, 0)),
            pl.BlockSpec((_BT, _K), lambda i: (i, 0)),
            full(cond_in, Wc.shape[1]),
            full(1, bc.shape[0]),
            full(_LATENT, _K * _HID),
            full(Wc.shape[1], _K * _HID),
            full(1, _K * _HID),
            full(1, _K * _HID),
            full(_K, _HID, _HID),
            full(_K, 1, _HID),
            full(_K, _HID, _LATENT),
            full(_K, _LATENT),
            full(_LATENT, out_dim),
            full(1, out_dim),
        ],
        out_specs=pl.BlockSpec((_BT, out_dim), lambda i: (i, 0)),
        out_shape=jax.ShapeDtypeStruct((b, out_dim), jnp.float32),
        compiler_params=pltpu.CompilerParams(
            dimension_semantics=("arbitrary",)),
    )(x_in, p_hat, Wc, bc.reshape(1, -1), W1u, W1c, w1tau, b1f, W2,
      b2.reshape(_K, 1, _HID), W3, b3, Wd, bd.reshape(1, -1))

    return out.reshape(b, _TA, _DA)
